# Initial kernel scaffold; baseline (speedup 1.0000x reference)
import jax
import jax.numpy as jnp
from jax.experimental import pallas as pl


def kernel(feat, subdet_id, proj_w, proj_b, type_table):
    raise NotImplementedError("write your pallas kernel here")


# speedup vs baseline: 4.6992x; 4.6992x over previous
import jax
import jax.numpy as jnp
from jax.experimental import pallas as pl
from jax.experimental.pallas import tpu as pltpu

_M = 1024


def _tile_body(ids_ref, x_ref, w_ref, tb_ref, out_ref):
    x = x_ref[...]
    ids = ids_ref[0, 0, :]
    n_sub = w_ref.shape[0]
    oh = (ids[:, None] == jax.lax.broadcasted_iota(jnp.int32, (1, n_sub), 1)
          ).astype(jnp.float32)
    acc = jnp.dot(oh, tb_ref[...], preferred_element_type=jnp.float32)
    for s in range(n_sub):
        xs = x * oh[:, s][:, None]
        acc = acc + jnp.dot(xs, w_ref[s], preferred_element_type=jnp.float32)
    out_ref[...] = acc


def kernel(feat, subdet_id, proj_w, proj_b, type_table):
    n, in_f = feat.shape
    n_sub, embed = type_table.shape
    ids3 = subdet_id.reshape(n // _M, 1, _M)
    tb = proj_b + type_table
    return pl.pallas_call(
        _tile_body,
        grid=(n // _M,),
        in_specs=[
            pl.BlockSpec((1, 1, _M), lambda i: (i, 0, 0)),
            pl.BlockSpec((_M, in_f), lambda i: (i, 0)),
            pl.BlockSpec((n_sub, in_f, embed), lambda i: (0, 0, 0)),
            pl.BlockSpec((n_sub, embed), lambda i: (0, 0)),
        ],
        out_specs=pl.BlockSpec((_M, embed), lambda i: (i, 0)),
        out_shape=jax.ShapeDtypeStruct((n, embed), jnp.float32),
        compiler_params=pltpu.CompilerParams(
            dimension_semantics=("arbitrary",)),
    )(ids3, feat, proj_w, tb)
